# baseline (speedup 1.0000x reference)
import jax
import jax.numpy as jnp
from jax.experimental import pallas as pl
from jax.experimental.pallas import tpu as pltpu

_TOKENS = 16384
_IN_FEATURES = 2048
_NUM_EXPERTS = 64
_TOP_K = 8
_BLOCK_T = 512


def _router_block(x_ref, w_ref, scores_ref, topk_s_ref, topk_i_ref):
    x = x_ref[...]
    w = w_ref[...]
    logits = jax.lax.dot_general(
        x, w, (((1,), (1,)), ((), ())), preferred_element_type=jnp.float32
    )
    m = jnp.max(logits, axis=1, keepdims=True)
    e = jnp.exp(logits - m)
    s = jnp.sum(e, axis=1, keepdims=True)
    scores = e / s
    scores_ref[...] = scores

    iota = jax.lax.broadcasted_iota(jnp.int32, scores.shape, 1)
    work = scores
    vals = []
    idxs = []
    for _ in range(_TOP_K):
        mv = jnp.max(work, axis=1, keepdims=True)
        am = jnp.min(
            jnp.where(work == mv, iota, _NUM_EXPERTS), axis=1, keepdims=True
        )
        vals.append(mv)
        idxs.append(am)
        work = jnp.where(iota == am, -1.0, work)
    v = jnp.concatenate(vals, axis=1)
    i = jnp.concatenate(idxs, axis=1)
    denom = jnp.sum(v, axis=1, keepdims=True) + 1e-6
    topk_s_ref[...] = v / denom
    topk_i_ref[...] = i


def kernel(x_video, W):
    grid = (_TOKENS // _BLOCK_T,)
    out_shapes = (
        jax.ShapeDtypeStruct((_TOKENS, _NUM_EXPERTS), jnp.float32),
        jax.ShapeDtypeStruct((_TOKENS, _TOP_K), jnp.float32),
        jax.ShapeDtypeStruct((_TOKENS, _TOP_K), jnp.int32),
    )
    scores, topk_scores, topk_idx = pl.pallas_call(
        _router_block,
        grid=grid,
        in_specs=[
            pl.BlockSpec((_BLOCK_T, _IN_FEATURES), lambda t: (t, 0)),
            pl.BlockSpec((_NUM_EXPERTS, _IN_FEATURES), lambda t: (0, 0)),
        ],
        out_specs=(
            pl.BlockSpec((_BLOCK_T, _NUM_EXPERTS), lambda t: (t, 0)),
            pl.BlockSpec((_BLOCK_T, _TOP_K), lambda t: (t, 0)),
            pl.BlockSpec((_BLOCK_T, _TOP_K), lambda t: (t, 0)),
        ),
        out_shape=out_shapes,
    )(x_video, W)
    return (scores, topk_scores, topk_idx)


# speedup vs baseline: 1.2057x; 1.2057x over previous
import jax
import jax.numpy as jnp
from jax.experimental import pallas as pl
from jax.experimental.pallas import tpu as pltpu

_TOKENS = 16384
_IN_FEATURES = 2048
_NUM_EXPERTS = 64
_TOP_K = 8
_BLOCK_T = 512


def _router_block(x_ref, w_ref, scores_ref, topk_s_ref, topk_i_ref):
    x = x_ref[...]
    w = w_ref[...]
    logits = jax.lax.dot_general(
        x, w, (((1,), (1,)), ((), ())), preferred_element_type=jnp.float32
    )
    m = jnp.max(logits, axis=1, keepdims=True)
    e = jnp.exp(logits - m)
    s = jnp.sum(e, axis=1, keepdims=True)
    scores_ref[...] = e / s

    iota = jax.lax.broadcasted_iota(jnp.int32, e.shape, 1)
    keys = (
        jax.lax.bitcast_convert_type(e, jnp.int32) & jnp.int32(~63)
    ) | (jnp.int32(_NUM_EXPERTS - 1) - iota)
    kvals = []
    for _ in range(_TOP_K):
        ki = jnp.max(keys, axis=1, keepdims=True)
        kvals.append(ki)
        keys = jnp.where(keys == ki, jnp.int32(-1), keys)
    k8 = jnp.concatenate(kvals, axis=1)
    v = jax.lax.bitcast_convert_type(k8 & jnp.int32(~63), jnp.float32)
    idx = jnp.int32(_NUM_EXPERTS - 1) - (k8 & jnp.int32(63))
    denom = jnp.sum(v, axis=1, keepdims=True) + 1e-6 * s
    topk_s_ref[...] = v / denom
    topk_i_ref[...] = idx


def kernel(x_video, W):
    grid = (_TOKENS // _BLOCK_T,)
    out_shapes = (
        jax.ShapeDtypeStruct((_TOKENS, _NUM_EXPERTS), jnp.float32),
        jax.ShapeDtypeStruct((_TOKENS, _TOP_K), jnp.float32),
        jax.ShapeDtypeStruct((_TOKENS, _TOP_K), jnp.int32),
    )
    scores, topk_scores, topk_idx = pl.pallas_call(
        _router_block,
        grid=grid,
        in_specs=[
            pl.BlockSpec((_BLOCK_T, _IN_FEATURES), lambda t: (t, 0)),
            pl.BlockSpec((_NUM_EXPERTS, _IN_FEATURES), lambda t: (0, 0)),
        ],
        out_specs=(
            pl.BlockSpec((_BLOCK_T, _NUM_EXPERTS), lambda t: (t, 0)),
            pl.BlockSpec((_BLOCK_T, _TOP_K), lambda t: (t, 0)),
            pl.BlockSpec((_BLOCK_T, _TOP_K), lambda t: (t, 0)),
        ),
        out_shape=out_shapes,
        compiler_params=pltpu.CompilerParams(
            dimension_semantics=("arbitrary",),
        ),
    )(x_video, W)
    return (scores, topk_scores, topk_idx)


# speedup vs baseline: 1.3590x; 1.1271x over previous
import jax
import jax.numpy as jnp
from jax.experimental import pallas as pl
from jax.experimental.pallas import tpu as pltpu

_TOKENS = 16384
_IN_FEATURES = 2048
_NUM_EXPERTS = 64
_TOP_K = 8
_BLOCK_T = 512


def _router_block(x_ref, w_ref, scores_ref, topk_s_ref, topk_i_ref):
    x = x_ref[...]
    w = w_ref[...]
    logits = jax.lax.dot_general(
        x, w, (((1,), (1,)), ((), ())), preferred_element_type=jnp.float32
    )
    m = jnp.max(logits, axis=1, keepdims=True)
    e = jnp.exp(logits - m)
    s = jnp.sum(e, axis=1, keepdims=True)
    scores_ref[...] = e / s

    iota = jax.lax.broadcasted_iota(jnp.int32, e.shape, 1)
    ikeys = (
        jax.lax.bitcast_convert_type(e, jnp.int32) & jnp.int32(~63)
    ) | (jnp.int32(_NUM_EXPERTS - 1) - iota)
    keys = jax.lax.bitcast_convert_type(ikeys, jnp.float32)
    kvals = []
    for _ in range(_TOP_K):
        ki = jnp.max(keys, axis=1, keepdims=True)
        kvals.append(ki)
        keys = jnp.where(keys == ki, jnp.float32(-1.0), keys)
    k8 = jax.lax.bitcast_convert_type(
        jnp.concatenate(kvals, axis=1), jnp.int32
    )
    v = jax.lax.bitcast_convert_type(k8 & jnp.int32(~63), jnp.float32)
    idx = jnp.int32(_NUM_EXPERTS - 1) - (k8 & jnp.int32(63))
    denom = jnp.sum(v, axis=1, keepdims=True) + 1e-6 * s
    topk_s_ref[...] = v / denom
    topk_i_ref[...] = idx


def kernel(x_video, W):
    grid = (_TOKENS // _BLOCK_T,)
    out_shapes = (
        jax.ShapeDtypeStruct((_TOKENS, _NUM_EXPERTS), jnp.float32),
        jax.ShapeDtypeStruct((_TOKENS, _TOP_K), jnp.float32),
        jax.ShapeDtypeStruct((_TOKENS, _TOP_K), jnp.int32),
    )
    scores, topk_scores, topk_idx = pl.pallas_call(
        _router_block,
        grid=grid,
        in_specs=[
            pl.BlockSpec((_BLOCK_T, _IN_FEATURES), lambda t: (t, 0)),
            pl.BlockSpec((_NUM_EXPERTS, _IN_FEATURES), lambda t: (0, 0)),
        ],
        out_specs=(
            pl.BlockSpec((_BLOCK_T, _NUM_EXPERTS), lambda t: (t, 0)),
            pl.BlockSpec((_BLOCK_T, _TOP_K), lambda t: (t, 0)),
            pl.BlockSpec((_BLOCK_T, _TOP_K), lambda t: (t, 0)),
        ),
        out_shape=out_shapes,
        compiler_params=pltpu.CompilerParams(
            dimension_semantics=("arbitrary",),
        ),
    )(x_video, W)
    return (scores, topk_scores, topk_idx)


# speedup vs baseline: 1.5558x; 1.1448x over previous
import jax
import jax.numpy as jnp
from jax.experimental import pallas as pl
from jax.experimental.pallas import tpu as pltpu

_TOKENS = 16384
_IN_FEATURES = 2048
_NUM_EXPERTS = 64
_TOP_K = 8
_BLOCK_T = 1024


def _router_block(x_ref, w_ref, scores_ref, topk_s_ref, topk_i_ref):
    x = x_ref[...]
    w = w_ref[...]
    logits = jax.lax.dot_general(
        x, w, (((1,), (1,)), ((), ())), preferred_element_type=jnp.float32
    )
    m = jnp.max(logits, axis=1, keepdims=True)
    e = jnp.exp(logits - m)
    s = jnp.sum(e, axis=1, keepdims=True)
    scores_ref[...] = e / s

    iota = jax.lax.broadcasted_iota(jnp.int32, e.shape, 1)
    ikeys = (
        jax.lax.bitcast_convert_type(e, jnp.int32) & jnp.int32(~63)
    ) | (jnp.int32(_NUM_EXPERTS - 1) - iota)
    keys = jax.lax.bitcast_convert_type(ikeys, jnp.float32)
    kvals = []
    for _ in range(_TOP_K):
        ki = jnp.max(keys, axis=1, keepdims=True)
        kvals.append(ki)
        keys = jnp.where(keys == ki, jnp.float32(-1.0), keys)
    k8 = jax.lax.bitcast_convert_type(
        jnp.concatenate(kvals, axis=1), jnp.int32
    )
    v = jax.lax.bitcast_convert_type(k8 & jnp.int32(~63), jnp.float32)
    idx = jnp.int32(_NUM_EXPERTS - 1) - (k8 & jnp.int32(63))
    denom = jnp.sum(v, axis=1, keepdims=True) + 1e-6 * s
    topk_s_ref[...] = v / denom
    topk_i_ref[...] = idx


def kernel(x_video, W):
    grid = (_TOKENS // _BLOCK_T,)
    out_shapes = (
        jax.ShapeDtypeStruct((_TOKENS, _NUM_EXPERTS), jnp.float32),
        jax.ShapeDtypeStruct((_TOKENS, _TOP_K), jnp.float32),
        jax.ShapeDtypeStruct((_TOKENS, _TOP_K), jnp.int32),
    )
    scores, topk_scores, topk_idx = pl.pallas_call(
        _router_block,
        grid=grid,
        in_specs=[
            pl.BlockSpec((_BLOCK_T, _IN_FEATURES), lambda t: (t, 0)),
            pl.BlockSpec((_NUM_EXPERTS, _IN_FEATURES), lambda t: (0, 0)),
        ],
        out_specs=(
            pl.BlockSpec((_BLOCK_T, _NUM_EXPERTS), lambda t: (t, 0)),
            pl.BlockSpec((_BLOCK_T, _TOP_K), lambda t: (t, 0)),
            pl.BlockSpec((_BLOCK_T, _TOP_K), lambda t: (t, 0)),
        ),
        out_shape=out_shapes,
        compiler_params=pltpu.CompilerParams(
            dimension_semantics=("arbitrary",),
        ),
    )(x_video, W)
    return (scores, topk_scores, topk_idx)


# speedup vs baseline: 1.6076x; 1.0333x over previous
import jax
import jax.numpy as jnp
from jax.experimental import pallas as pl
from jax.experimental.pallas import tpu as pltpu

_TOKENS = 16384
_IN_FEATURES = 2048
_NUM_EXPERTS = 64
_TOP_K = 8
_BLOCK_T = 2048


def _router_block(x_ref, w_ref, scores_ref, topk_s_ref, topk_i_ref):
    x = x_ref[...]
    w = w_ref[...]
    logits = jax.lax.dot_general(
        x, w, (((1,), (1,)), ((), ())), preferred_element_type=jnp.float32
    )
    m = jnp.max(logits, axis=1, keepdims=True)
    e = jnp.exp(logits - m)
    s = jnp.sum(e, axis=1, keepdims=True)
    scores_ref[...] = e / s

    iota = jax.lax.broadcasted_iota(jnp.int32, e.shape, 1)
    ikeys = (
        jax.lax.bitcast_convert_type(e, jnp.int32) & jnp.int32(~63)
    ) | (jnp.int32(_NUM_EXPERTS - 1) - iota)
    keys = jax.lax.bitcast_convert_type(ikeys, jnp.float32)
    kvals = []
    for _ in range(_TOP_K):
        ki = jnp.max(keys, axis=1, keepdims=True)
        kvals.append(ki)
        keys = jnp.where(keys == ki, jnp.float32(-1.0), keys)
    k8 = jax.lax.bitcast_convert_type(
        jnp.concatenate(kvals, axis=1), jnp.int32
    )
    v = jax.lax.bitcast_convert_type(k8 & jnp.int32(~63), jnp.float32)
    idx = jnp.int32(_NUM_EXPERTS - 1) - (k8 & jnp.int32(63))
    denom = jnp.sum(v, axis=1, keepdims=True) + 1e-6 * s
    topk_s_ref[...] = v / denom
    topk_i_ref[...] = idx


def kernel(x_video, W):
    grid = (_TOKENS // _BLOCK_T,)
    out_shapes = (
        jax.ShapeDtypeStruct((_TOKENS, _NUM_EXPERTS), jnp.float32),
        jax.ShapeDtypeStruct((_TOKENS, _TOP_K), jnp.float32),
        jax.ShapeDtypeStruct((_TOKENS, _TOP_K), jnp.int32),
    )
    scores, topk_scores, topk_idx = pl.pallas_call(
        _router_block,
        grid=grid,
        in_specs=[
            pl.BlockSpec((_BLOCK_T, _IN_FEATURES), lambda t: (t, 0)),
            pl.BlockSpec((_NUM_EXPERTS, _IN_FEATURES), lambda t: (0, 0)),
        ],
        out_specs=(
            pl.BlockSpec((_BLOCK_T, _NUM_EXPERTS), lambda t: (t, 0)),
            pl.BlockSpec((_BLOCK_T, _TOP_K), lambda t: (t, 0)),
            pl.BlockSpec((_BLOCK_T, _TOP_K), lambda t: (t, 0)),
        ),
        out_shape=out_shapes,
        compiler_params=pltpu.CompilerParams(
            dimension_semantics=("arbitrary",),
        ),
    )(x_video, W)
    return (scores, topk_scores, topk_idx)
